# baseline (speedup 1.0000x reference)
import jax
import jax.numpy as jnp
from jax import lax
from jax.experimental import pallas as pl
from jax.experimental.pallas import tpu as pltpu
from jax.experimental.pallas import tpu_sc as plsc

B = 16384
D = 64
V = 100000
HALF_V = 50176

NC = 2
NS = 16
NW = NC * NS
BPW = B // NW
IDX_CHUNK = 128
GBUF = 256

TK = 512
NBH = HALF_V // TK

BLK = 2048


def _repack_body(x1_ref, x2_ref, o_ref):
  o_ref[...] = jnp.concatenate([x1_ref[...].T, x2_ref[...].T], axis=1)


def _repack(tab_t):
  return pl.pallas_call(
      _repack_body,
      grid=(NBH,),
      in_specs=[
          pl.BlockSpec((D, TK), lambda n: (0, n)),
          pl.BlockSpec((D, TK), lambda n: (0, NBH + n)),
      ],
      out_specs=pl.BlockSpec((TK, 2 * D), lambda n: (n, 0)),
      out_shape=jax.ShapeDtypeStruct((HALF_V, 2 * D), jnp.float32),
      compiler_params=pltpu.CompilerParams(
          dimension_semantics=("arbitrary",),
      ),
  )(tab_t, tab_t)


def _gather_body(idx_h, tab_h, out_h, idx_v, buf, sem):
  wid = lax.axis_index("s") * NC + lax.axis_index("c")
  base = wid * BPW
  pltpu.sync_copy(idx_h.at[pl.ds(base, BPW)], idx_v)
  for g in range(BPW // GBUF):
    copies = []
    for j in range(GBUF // IDX_CHUNK):
      isl = pl.ds(g * GBUF + j * IDX_CHUNK, IDX_CHUNK)
      dsl = pl.ds(j * IDX_CHUNK, IDX_CHUNK)
      copies.append(pltpu.async_copy(tab_h.at[idx_v.at[isl]], buf.at[dsl], sem))
    for c in copies:
      c.wait()
    pltpu.sync_copy(buf, out_h.at[pl.ds(base + g * GBUF, GBUF)])


def _sc_gather(idx_half, tab2):
  mesh = plsc.VectorSubcoreMesh(core_axis_name="c", subcore_axis_name="s")
  f = pl.kernel(
      _gather_body,
      out_type=jax.ShapeDtypeStruct((B, 2 * D), jnp.float32),
      mesh=mesh,
      scratch_types=[
          pltpu.VMEM((BPW,), jnp.int32),
          pltpu.VMEM((GBUF, 2 * D), jnp.float32),
          pltpu.SemaphoreType.DMA,
      ],
  )
  return f(idx_half, tab2)


def _mlp_body(u2_ref, i2_ref, pu_ref, pi_ref, wa_ref, wb_ref, wc_ref,
              b1_ref, w2t_ref, b2_ref, o_ref):
  u2 = u2_ref[...]
  i2 = i2_ref[...]
  u = jnp.where(pu_ref[...] > 0.5, u2[:, D:2 * D], u2[:, 0:D])
  v = jnp.where(pi_ref[...] > 0.5, i2[:, D:2 * D], i2[:, 0:D])
  e = u * v
  h = (jnp.dot(e, wa_ref[...], preferred_element_type=jnp.float32)
       + jnp.dot(u, wb_ref[...], preferred_element_type=jnp.float32)
       + jnp.dot(v, wc_ref[...], preferred_element_type=jnp.float32)
       + b1_ref[...])
  h = jnp.maximum(h, 0.0)
  z = jnp.sum(h * w2t_ref[...], axis=1, keepdims=True) + b2_ref[...]
  o_ref[...] = jax.nn.sigmoid(z)


def _tc_mlp(u2, i2, pu, pi, W1, b1, W2, b2):
  wa = W1[0:D]
  wb = W1[D:2 * D]
  wc = W1[2 * D:3 * D]
  b1r = b1.reshape(1, 8)
  w2t = W2.reshape(1, 8)
  b2r = b2.reshape(1, 1)
  grid = (B // BLK,)
  return pl.pallas_call(
      _mlp_body,
      grid=grid,
      in_specs=[
          pl.BlockSpec((BLK, 2 * D), lambda n: (n, 0)),
          pl.BlockSpec((BLK, 2 * D), lambda n: (n, 0)),
          pl.BlockSpec((BLK, 1), lambda n: (n, 0)),
          pl.BlockSpec((BLK, 1), lambda n: (n, 0)),
          pl.BlockSpec((D, 8), lambda n: (0, 0)),
          pl.BlockSpec((D, 8), lambda n: (0, 0)),
          pl.BlockSpec((D, 8), lambda n: (0, 0)),
          pl.BlockSpec((1, 8), lambda n: (0, 0)),
          pl.BlockSpec((1, 8), lambda n: (0, 0)),
          pl.BlockSpec((1, 1), lambda n: (0, 0)),
      ],
      out_specs=pl.BlockSpec((BLK, 1), lambda n: (n, 0)),
      out_shape=jax.ShapeDtypeStruct((B, 1), jnp.float32),
      compiler_params=pltpu.CompilerParams(
          dimension_semantics=("arbitrary",),
      ),
  )(u2, i2, pu, pi, wa, wb, wc, b1r, w2t, b2r)


@jax.jit
def kernel(group_inputs, user_inputs, item_inputs, user_table, item_table, W1, b1, W2, b2):
  del group_inputs
  ui = user_inputs.astype(jnp.int32)
  ii = item_inputs.astype(jnp.int32)
  utab2 = _repack(user_table.T)
  itab2 = _repack(item_table.T)
  u2 = _sc_gather(ui % HALF_V, utab2)
  i2 = _sc_gather(ii % HALF_V, itab2)
  pu = (ui >= HALF_V).astype(jnp.float32).reshape(B, 1)
  pi = (ii >= HALF_V).astype(jnp.float32).reshape(B, 1)
  return _tc_mlp(u2, i2, pu, pi, W1, b1, W2, b2)


# speedup vs baseline: 1.2788x; 1.2788x over previous
import jax
import jax.numpy as jnp
from jax import lax
from jax.experimental import pallas as pl
from jax.experimental.pallas import tpu as pltpu
from jax.experimental.pallas import tpu_sc as plsc

B = 16384
D = 64
V = 100000
HALF_V = 50176

NC = 2
NS = 16
NW = NC * NS
BPW = B // NW
IDX_CHUNK = 128
GBUF = 256

TK = 1024
NBH = HALF_V // TK

BLK = 2048

_T_DIMS = (((0,), (0,)), ((), ()))


def _repack_body(x1_ref, x2_ref, eye_ref, o_ref):
  eye = eye_ref[...]
  t1 = lax.dot_general(x1_ref[...], eye, _T_DIMS,
                       preferred_element_type=jnp.float32)
  t2 = lax.dot_general(x2_ref[...], eye, _T_DIMS,
                       preferred_element_type=jnp.float32)
  o_ref[...] = jnp.concatenate([t1, t2], axis=1)


def _repack(tab_t, eye):
  return pl.pallas_call(
      _repack_body,
      grid=(NBH,),
      in_specs=[
          pl.BlockSpec((D, TK), lambda n: (0, n)),
          pl.BlockSpec((D, TK), lambda n: (0, NBH + n)),
          pl.BlockSpec((D, D), lambda n: (0, 0)),
      ],
      out_specs=pl.BlockSpec((TK, 2 * D), lambda n: (n, 0)),
      out_shape=jax.ShapeDtypeStruct((HALF_V, 2 * D), jnp.float32),
      compiler_params=pltpu.CompilerParams(
          dimension_semantics=("arbitrary",),
      ),
  )(tab_t, tab_t, eye)


def _gather_body(idx_h, tab_h, out_h, idx_v, buf, sem):
  wid = lax.axis_index("s") * NC + lax.axis_index("c")
  base = wid * BPW
  pltpu.sync_copy(idx_h.at[pl.ds(base, BPW)], idx_v)
  for g in range(BPW // GBUF):
    copies = []
    for j in range(GBUF // IDX_CHUNK):
      isl = pl.ds(g * GBUF + j * IDX_CHUNK, IDX_CHUNK)
      dsl = pl.ds(j * IDX_CHUNK, IDX_CHUNK)
      copies.append(pltpu.async_copy(tab_h.at[idx_v.at[isl]], buf.at[dsl], sem))
    for c in copies:
      c.wait()
    pltpu.sync_copy(buf, out_h.at[pl.ds(base + g * GBUF, GBUF)])


def _sc_gather(idx_half, tab2):
  mesh = plsc.VectorSubcoreMesh(core_axis_name="c", subcore_axis_name="s")
  f = pl.kernel(
      _gather_body,
      out_type=jax.ShapeDtypeStruct((B, 2 * D), jnp.float32),
      mesh=mesh,
      scratch_types=[
          pltpu.VMEM((BPW,), jnp.int32),
          pltpu.VMEM((GBUF, 2 * D), jnp.float32),
          pltpu.SemaphoreType.DMA,
      ],
  )
  return f(idx_half, tab2)


def _mlp_body(u2_ref, i2_ref, pu_ref, pi_ref, wa_ref, wb_ref, wc_ref,
              b1_ref, w2t_ref, b2_ref, o_ref):
  u2 = u2_ref[...]
  i2 = i2_ref[...]
  u = jnp.where(pu_ref[...] > 0.5, u2[:, D:2 * D], u2[:, 0:D])
  v = jnp.where(pi_ref[...] > 0.5, i2[:, D:2 * D], i2[:, 0:D])
  e = u * v
  h = (jnp.dot(e, wa_ref[...], preferred_element_type=jnp.float32)
       + jnp.dot(u, wb_ref[...], preferred_element_type=jnp.float32)
       + jnp.dot(v, wc_ref[...], preferred_element_type=jnp.float32)
       + b1_ref[...])
  h = jnp.maximum(h, 0.0)
  z = jnp.sum(h * w2t_ref[...], axis=1, keepdims=True) + b2_ref[...]
  o_ref[...] = jax.nn.sigmoid(z)


def _tc_mlp(u2, i2, pu, pi, W1, b1, W2, b2):
  wa = W1[0:D]
  wb = W1[D:2 * D]
  wc = W1[2 * D:3 * D]
  b1r = b1.reshape(1, 8)
  w2t = W2.reshape(1, 8)
  b2r = b2.reshape(1, 1)
  grid = (B // BLK,)
  return pl.pallas_call(
      _mlp_body,
      grid=grid,
      in_specs=[
          pl.BlockSpec((BLK, 2 * D), lambda n: (n, 0)),
          pl.BlockSpec((BLK, 2 * D), lambda n: (n, 0)),
          pl.BlockSpec((BLK, 1), lambda n: (n, 0)),
          pl.BlockSpec((BLK, 1), lambda n: (n, 0)),
          pl.BlockSpec((D, 8), lambda n: (0, 0)),
          pl.BlockSpec((D, 8), lambda n: (0, 0)),
          pl.BlockSpec((D, 8), lambda n: (0, 0)),
          pl.BlockSpec((1, 8), lambda n: (0, 0)),
          pl.BlockSpec((1, 8), lambda n: (0, 0)),
          pl.BlockSpec((1, 1), lambda n: (0, 0)),
      ],
      out_specs=pl.BlockSpec((BLK, 1), lambda n: (n, 0)),
      out_shape=jax.ShapeDtypeStruct((B, 1), jnp.float32),
      compiler_params=pltpu.CompilerParams(
          dimension_semantics=("arbitrary",),
      ),
  )(u2, i2, pu, pi, wa, wb, wc, b1r, w2t, b2r)


@jax.jit
def kernel(group_inputs, user_inputs, item_inputs, user_table, item_table, W1, b1, W2, b2):
  del group_inputs
  ui = user_inputs.astype(jnp.int32)
  ii = item_inputs.astype(jnp.int32)
  eye = jnp.eye(D, dtype=jnp.float32)
  utab2 = _repack(user_table.T, eye)
  itab2 = _repack(item_table.T, eye)
  u2 = _sc_gather(ui % HALF_V, utab2)
  i2 = _sc_gather(ii % HALF_V, itab2)
  pu = (ui >= HALF_V).astype(jnp.float32).reshape(B, 1)
  pi = (ii >= HALF_V).astype(jnp.float32).reshape(B, 1)
  return _tc_mlp(u2, i2, pu, pi, W1, b1, W2, b2)
